# Initial kernel scaffold; baseline (speedup 1.0000x reference)
import jax
import jax.numpy as jnp
from jax.experimental import pallas as pl


def kernel(x, edge_index, edge_attr, all_node_features, rel_features, paths, links, x_lin2_W, x_lin2_b, nn_cd_W, nn_cd_b, lstm_Wih, lstm_Whh, lstm_b, conv_W, conv_b, lin1_W, lin1_b, lin2_W, lin2_b, prelu_a):
    raise NotImplementedError("write your pallas kernel here")


# speedup vs baseline: 5.8092x; 5.8092x over previous
import functools

import jax
import jax.numpy as jnp
from jax import lax
from jax.experimental import pallas as pl
from jax.experimental.pallas import tpu as pltpu
from jax.experimental.pallas import tpu_sc as plsc

F32 = jnp.float32
I32 = jnp.int32

N_NODES = 4000
NP = 4096
SH = 4224
DUMMY = 4096
NFEAT = 128
HID = 128
CONV = 64
T = 7
P = 8000
PP = 8192
FLAT = PP * T
E = 128000
EP = 131072
L = 10000
LP = 10240
NW = 32

_mesh = lambda: plsc.VectorSubcoreMesh(core_axis_name="c", subcore_axis_name="s")


@functools.cache
def _mk_gather(D, R, rw, label):
    @functools.partial(
        pl.kernel,
        out_type=jax.ShapeDtypeStruct((R * 128, D), F32),
        mesh=_mesh(),
        scratch_types=[
            pltpu.VMEM((rw, 128), I32),
            pltpu.VMEM((128, D), F32),
            pltpu.SemaphoreType.DMA,
        ],
        name=label,
    )
    def k(table, idx, out, idx_v, rows_v, sem):
        c = lax.axis_index("c")
        s = lax.axis_index("s")
        wid = s * 2 + c
        base = wid * rw
        pltpu.sync_copy(idx.at[wid], idx_v)

        def body(j, carry):
            pltpu.async_copy(table.at[idx_v.at[j]], rows_v, sem).wait()
            pltpu.sync_copy(rows_v, out.at[pl.ds((base + j) * 128, 128)])
            return carry

        lax.fori_loop(0, rw, body, 0)

    return k


def _gather_paths(table, idx):
    return _mk_gather(NFEAT, FLAT // 128, FLAT // 128 // NW, "path_gather")(table, idx)


def _gather_links(table, idx):
    return _mk_gather(256, (2 * LP) // 128, (2 * LP) // 128 // NW, "link_gather")(table, idx)


_SROWS = SH // 16


@functools.cache
def _mk_edge_scatter():
    @functools.partial(
        pl.kernel,
        out_type=jax.ShapeDtypeStruct((2, SH, CONV), F32),
        mesh=_mesh(),
        scratch_types=[
            pltpu.VMEM((EP // 128 // NW, 128), I32),
            pltpu.VMEM((EP // 128 // NW, 128), I32),
            pltpu.VMEM((128, CONV), F32),
            pltpu.VMEM_SHARED((SH, CONV), F32),
            pltpu.SemaphoreType.DMA,
        ],
        compiler_params=pltpu.CompilerParams(use_tc_tiling_on_sc=False),
        name="edge_scatter",
    )
    def k(xw, sI, dI, z, out, sv, dv, rows, shared, sem):
        c = lax.axis_index("c")
        s = lax.axis_index("s")
        wid = s * 2 + c
        nchunk = EP // 128 // NW
        pltpu.sync_copy(z.at[pl.ds(s * _SROWS, _SROWS)], shared.at[pl.ds(s * _SROWS, _SROWS)])
        pltpu.sync_copy(sI.at[wid], sv)
        pltpu.sync_copy(dI.at[wid], dv)
        plsc.subcore_barrier()

        def body(j, carry):
            pltpu.async_copy(xw.at[sv.at[j]], rows, sem).wait()
            pltpu.sync_copy(rows, shared.at[dv.at[j]], add=True)
            return carry

        lax.fori_loop(0, nchunk, body, 0)
        plsc.subcore_barrier()
        pltpu.sync_copy(shared.at[pl.ds(s * _SROWS, _SROWS)], out.at[c, pl.ds(s * _SROWS, _SROWS)])

    return k


def _edge_scatter(xw, sI, dI, z):
    return _mk_edge_scatter()(xw, sI, dI, z)


@functools.cache
def _mk_deg_hist():
    @functools.partial(
        pl.kernel,
        out_type=jax.ShapeDtypeStruct((2, SH, 16), F32),
        mesh=_mesh(),
        scratch_types=[
            pltpu.VMEM((EP // 128 // NW, 128), I32),
            pltpu.VMEM((128, 16), F32),
            pltpu.VMEM_SHARED((SH, 16), F32),
            pltpu.SemaphoreType.DMA,
        ],
        compiler_params=pltpu.CompilerParams(use_tc_tiling_on_sc=False),
        name="deg_hist",
    )
    def k(dI, z, ones, out, dv, ones_v, shared, sem):
        c = lax.axis_index("c")
        s = lax.axis_index("s")
        wid = s * 2 + c
        nchunk = EP // 128 // NW
        pltpu.sync_copy(z.at[pl.ds(s * _SROWS, _SROWS)], shared.at[pl.ds(s * _SROWS, _SROWS)])
        pltpu.sync_copy(dI.at[wid], dv)
        pltpu.sync_copy(ones, ones_v)
        plsc.subcore_barrier()

        def body(j, carry):
            pltpu.sync_copy(ones_v, shared.at[dv.at[j]], add=True)
            return carry

        lax.fori_loop(0, nchunk, body, 0)
        plsc.subcore_barrier()
        pltpu.sync_copy(shared.at[pl.ds(s * _SROWS, _SROWS)], out.at[c, pl.ds(s * _SROWS, _SROWS)])

    return k


def _deg_hist(dI, z, ones):
    return _mk_deg_hist()(dI, z, ones)


_PB = 512
_NB = 256
_GRID = PP // _PB


def _lstm_body(f0, f1, f2, f3, f4, f5, f6, wih, whh, b, cw, c0, c1, a, out):
    fs = (f0, f1, f2, f3, f4, f5, f6)
    W_ih = wih[:]
    W_hh = whh[:]
    bb = b[:]
    h = jnp.zeros((_PB, HID), F32)
    c = jnp.zeros((_PB, HID), F32)
    for t in range(T):
        xt = fs[t][:]
        g = (jnp.dot(xt, W_ih, preferred_element_type=F32)
             + jnp.dot(h, W_hh, preferred_element_type=F32) + bb)
        ig = jax.nn.sigmoid(g[:, :HID])
        fg = jax.nn.sigmoid(g[:, HID:2 * HID])
        gg = jnp.tanh(g[:, 2 * HID:3 * HID])
        og = jax.nn.sigmoid(g[:, 3 * HID:])
        c = fg * c + ig * gg
        h = og * jnp.tanh(c)
    hr = h.reshape(_NB, 2 * HID)
    hm = (hr[:, :HID] + hr[:, HID:]) * 0.5
    av = a[0, 0]
    hp = jnp.where(hm > 0, hm, av * hm)
    xw = jnp.dot(hp, cw[:], preferred_element_type=F32)
    dinv = lax.rsqrt(c0[:, :1] + c1[:, :1] + 1.0)
    out[:] = xw * dinv


_lstm_call = pl.pallas_call(
    _lstm_body,
    grid=(_GRID,),
    in_specs=[pl.BlockSpec((_PB, NFEAT), functools.partial(lambda i, t: (t * _GRID + i, 0), t=t))
              for t in range(T)]
    + [
        pl.BlockSpec((NFEAT, 4 * HID), lambda i: (0, 0)),
        pl.BlockSpec((HID, 4 * HID), lambda i: (0, 0)),
        pl.BlockSpec((1, 4 * HID), lambda i: (0, 0)),
        pl.BlockSpec((HID, CONV), lambda i: (0, 0)),
        pl.BlockSpec((_NB, 16), lambda i: (i, 0)),
        pl.BlockSpec((_NB, 16), lambda i: (i, 0)),
        pl.BlockSpec(memory_space=pltpu.SMEM),
    ],
    out_specs=pl.BlockSpec((_NB, CONV), lambda i: (i, 0)),
    out_shape=jax.ShapeDtypeStruct((NP, CONV), F32),
)


def _fin_body(s0, s1, xw, c0, c1, b, out):
    acc = s0[:] + s1[:] + xw[:]
    dinv = lax.rsqrt(c0[:, :1] + c1[:, :1] + 1.0)
    v = acc * dinv + b[:]
    n = jnp.sqrt(jnp.sum(v * v, axis=1, keepdims=True))
    out[:] = v / jnp.maximum(n, 1e-12)


_fin_call = pl.pallas_call(
    _fin_body,
    grid=(4,),
    in_specs=[
        pl.BlockSpec((1024, CONV), lambda i: (i, 0)),
        pl.BlockSpec((1024, CONV), lambda i: (i, 0)),
        pl.BlockSpec((1024, CONV), lambda i: (i, 0)),
        pl.BlockSpec((1024, 16), lambda i: (i, 0)),
        pl.BlockSpec((1024, 16), lambda i: (i, 0)),
        pl.BlockSpec((1, CONV), lambda i: (0, 0)),
    ],
    out_specs=pl.BlockSpec((1024, CONV), lambda i: (i, 0)),
    out_shape=jax.ShapeDtypeStruct((NP, CONV), F32),
)


_LB = 512


def _score_body(ga, gb, w1a, w1b, b1, w2, b2, a, out):
    h = (jnp.dot(ga[:], w1a[:], preferred_element_type=F32)
         + jnp.dot(gb[:], w1b[:], preferred_element_type=F32) + b1[:])
    av = a[0, 0]
    h = jnp.where(h > 0, h, av * h)
    out[:] = jnp.dot(h, w2[:], preferred_element_type=F32) + b2[0, 0]


_score_call = pl.pallas_call(
    _score_body,
    grid=(LP // _LB,),
    in_specs=[
        pl.BlockSpec((_LB, 256), lambda i: (i, 0)),
        pl.BlockSpec((_LB, 256), lambda i: (i + LP // _LB, 0)),
        pl.BlockSpec((256, 256), lambda i: (0, 0)),
        pl.BlockSpec((256, 256), lambda i: (0, 0)),
        pl.BlockSpec((1, 256), lambda i: (0, 0)),
        pl.BlockSpec((256, 1), lambda i: (0, 0)),
        pl.BlockSpec(memory_space=pltpu.SMEM),
        pl.BlockSpec(memory_space=pltpu.SMEM),
    ],
    out_specs=pl.BlockSpec((_LB, 1), lambda i: (i, 0)),
    out_shape=jax.ShapeDtypeStruct((LP, 1), F32),
)


def kernel(x, edge_index, edge_attr, all_node_features, rel_features, paths, links,
           x_lin2_W, x_lin2_b, nn_cd_W, nn_cd_b, lstm_Wih, lstm_Whh, lstm_b,
           conv_W, conv_b, lin1_W, lin1_b, lin2_W, lin2_b, prelu_a):
    x_all = jnp.concatenate([all_node_features, rel_features], axis=0)
    src = edge_index[0]
    dst = edge_index[1]
    srcp = jnp.concatenate([src, jnp.zeros((EP - E,), I32)]).reshape(NW, -1, 128)
    dstp = jnp.concatenate([dst, jnp.full((EP - E,), DUMMY, I32)]).reshape(NW, -1, 128)
    pp = jnp.concatenate([paths, jnp.zeros((PP - P, T), I32)], axis=0)
    pidx = pp.T.reshape(NW, -1, 128)
    zeros64 = jnp.zeros((SH, CONV), F32)
    zeros16 = jnp.zeros((SH, 16), F32)
    ones16 = jnp.zeros((128, 16), F32).at[:, 0].set(1.0)
    a2 = prelu_a.reshape(1, 1)

    cnt = _deg_hist(dstp, zeros16, ones16)
    c0 = cnt[0, :NP]
    c1 = cnt[1, :NP]

    states = []
    for i in range(4):
        feats = _gather_paths(x_all, pidx)
        xwp = _lstm_call(*[feats] * T, lstm_Wih[i], lstm_Whh[i],
                         lstm_b[i].reshape(1, -1), conv_W[i], c0, c1, a2)
        S = _edge_scatter(xwp, srcp, dstp, zeros64)
        xc4 = _fin_call(S[0, :NP], S[1, :NP], xwp, c0, c1, conv_b[i].reshape(1, -1))
        states.append(xc4)
        x_pad = jnp.pad(xc4[:N_NODES], ((0, 0), (0, NFEAT - CONV)))
        x_all = x_all.at[2 * N_NODES:3 * N_NODES].set(x_pad)

    cs4 = jnp.concatenate(states, axis=1)
    cs = cs4[:N_NODES]

    l0 = jnp.pad(links[0], (0, LP - L))
    l1 = jnp.pad(links[1], (0, LP - L)) + 2000
    lidx = jnp.concatenate([l0, l1]).reshape(NW, -1, 128)
    g = _gather_links(cs4, lidx)
    outp = _score_call(g, g, lin1_W[:256], lin1_W[256:], lin1_b.reshape(1, -1),
                       lin2_W, lin2_b.reshape(1, 1), a2)
    out = outp[:L, 0]
    return (out, cs, x_all)


# speedup vs baseline: 6.2606x; 1.0777x over previous
import functools

import jax
import jax.numpy as jnp
from jax import lax
from jax.experimental import pallas as pl
from jax.experimental.pallas import tpu as pltpu
from jax.experimental.pallas import tpu_sc as plsc

F32 = jnp.float32
I32 = jnp.int32

N_NODES = 4000
NP = 4096
SH = 4224
DUMMY = 4096
NFEAT = 128
HID = 128
CONV = 64
T = 7
P = 8000
PP = 8192
FLAT = PP * T
E = 128000
EP = 131072
L = 10000
LP = 10240
NW = 32

_mesh = lambda: plsc.VectorSubcoreMesh(core_axis_name="c", subcore_axis_name="s")


@functools.cache
def _mk_gather(D, R, rw, ring, label):
    depth = ring // 2

    @functools.partial(
        pl.kernel,
        out_type=jax.ShapeDtypeStruct((R * 128, D), F32),
        mesh=_mesh(),
        scratch_types=[
            pltpu.VMEM((rw, 128), I32),
            pltpu.VMEM((ring * 128, D), F32),
            pltpu.SemaphoreType.DMA,
            pltpu.SemaphoreType.DMA,
        ],
        name=label,
    )
    def k(table, idx, out, idx_v, rows_v, gsem, wsem):
        c = lax.axis_index("c")
        s = lax.axis_index("s")
        wid = s * 2 + c
        base = wid * rw
        pltpu.sync_copy(idx.at[wid], idx_v)

        def buf(j):
            return rows_v.at[pl.ds((j % ring) * 128, 128)]

        gd = {}
        wd = {}
        for j in range(min(depth, rw)):
            gd[j] = pltpu.async_copy(table.at[idx_v.at[j]], buf(j), gsem)
        for j in range(rw):
            gd[j].wait()
            wd[j] = pltpu.async_copy(buf(j), out.at[pl.ds((base + j) * 128, 128)], wsem)
            if j >= depth:
                wd[j - depth].wait()
            if j + depth < rw:
                gd[j + depth] = pltpu.async_copy(
                    table.at[idx_v.at[j + depth]], buf(j + depth), gsem)
        for j in range(max(rw - depth, 0), rw):
            wd[j].wait()

    return k


def _gather_paths(table, idx):
    return _mk_gather(NFEAT, FLAT // 128, FLAT // 128 // NW, 4, "path_gather")(table, idx)


def _gather_links(table, idx):
    return _mk_gather(256, (2 * LP) // 128, (2 * LP) // 128 // NW, 2, "link_gather")(table, idx)


_SROWS = SH // 16


@functools.cache
def _mk_edge_scatter():
    @functools.partial(
        pl.kernel,
        out_type=jax.ShapeDtypeStruct((2, SH, CONV), F32),
        mesh=_mesh(),
        scratch_types=[
            pltpu.VMEM((EP // 128 // NW, 128), I32),
            pltpu.VMEM((EP // 128 // NW, 128), I32),
            pltpu.VMEM((8 * 128, CONV), F32),
            pltpu.VMEM_SHARED((SH, CONV), F32),
            pltpu.SemaphoreType.DMA,
            pltpu.SemaphoreType.DMA,
        ],
        compiler_params=pltpu.CompilerParams(use_tc_tiling_on_sc=False),
        name="edge_scatter",
    )
    def k(xw, sI, dI, z, out, sv, dv, rows, shared, gsem, ssem):
        c = lax.axis_index("c")
        s = lax.axis_index("s")
        wid = s * 2 + c
        nchunk = EP // 128 // NW
        ring, depth = 8, 4
        pltpu.sync_copy(z.at[pl.ds(s * _SROWS, _SROWS)], shared.at[pl.ds(s * _SROWS, _SROWS)])
        pltpu.sync_copy(sI.at[wid], sv)
        pltpu.sync_copy(dI.at[wid], dv)
        plsc.subcore_barrier()

        def buf(j):
            return rows.at[pl.ds((j % ring) * 128, 128)]

        gd = {}
        sd = {}
        for j in range(depth):
            gd[j] = pltpu.async_copy(xw.at[sv.at[j]], buf(j), gsem)
        for j in range(nchunk):
            gd[j].wait()
            sd[j] = pltpu.async_copy(buf(j), shared.at[dv.at[j]], ssem, add=True)
            if j >= depth:
                sd[j - depth].wait()
            if j + depth < nchunk:
                gd[j + depth] = pltpu.async_copy(
                    xw.at[sv.at[j + depth]], buf(j + depth), gsem)
        for j in range(nchunk - depth, nchunk):
            sd[j].wait()
        plsc.subcore_barrier()
        pltpu.sync_copy(shared.at[pl.ds(s * _SROWS, _SROWS)], out.at[c, pl.ds(s * _SROWS, _SROWS)])

    return k


def _edge_scatter(xw, sI, dI, z):
    return _mk_edge_scatter()(xw, sI, dI, z)


@functools.cache
def _mk_deg_hist():
    @functools.partial(
        pl.kernel,
        out_type=jax.ShapeDtypeStruct((2, SH, 16), F32),
        mesh=_mesh(),
        scratch_types=[
            pltpu.VMEM((EP // 128 // NW, 128), I32),
            pltpu.VMEM((128, 16), F32),
            pltpu.VMEM_SHARED((SH, 16), F32),
            pltpu.SemaphoreType.DMA,
        ],
        compiler_params=pltpu.CompilerParams(use_tc_tiling_on_sc=False),
        name="deg_hist",
    )
    def k(dI, z, ones, out, dv, ones_v, shared, sem):
        c = lax.axis_index("c")
        s = lax.axis_index("s")
        wid = s * 2 + c
        nchunk = EP // 128 // NW
        pltpu.sync_copy(z.at[pl.ds(s * _SROWS, _SROWS)], shared.at[pl.ds(s * _SROWS, _SROWS)])
        pltpu.sync_copy(dI.at[wid], dv)
        pltpu.sync_copy(ones, ones_v)
        plsc.subcore_barrier()

        sd = {}
        for j in range(nchunk):
            sd[j] = pltpu.async_copy(ones_v, shared.at[dv.at[j]], sem, add=True)
            if j >= 8:
                sd[j - 8].wait()
        for j in range(nchunk - 8, nchunk):
            sd[j].wait()
        plsc.subcore_barrier()
        pltpu.sync_copy(shared.at[pl.ds(s * _SROWS, _SROWS)], out.at[c, pl.ds(s * _SROWS, _SROWS)])

    return k


def _deg_hist(dI, z, ones):
    return _mk_deg_hist()(dI, z, ones)


_PB = 512
_NB = 256
_GRID = PP // _PB


def _lstm_body(f0, f1, f2, f3, f4, f5, f6, wih, whh, b, cw, c0, c1, a, out):
    fs = (f0, f1, f2, f3, f4, f5, f6)
    W_ih = wih[:]
    W_hh = whh[:]
    bb = b[:]
    h = jnp.zeros((_PB, HID), F32)
    c = jnp.zeros((_PB, HID), F32)
    for t in range(T):
        xt = fs[t][:]
        g = (jnp.dot(xt, W_ih, preferred_element_type=F32)
             + jnp.dot(h, W_hh, preferred_element_type=F32) + bb)
        ig = jax.nn.sigmoid(g[:, :HID])
        fg = jax.nn.sigmoid(g[:, HID:2 * HID])
        gg = jnp.tanh(g[:, 2 * HID:3 * HID])
        og = jax.nn.sigmoid(g[:, 3 * HID:])
        c = fg * c + ig * gg
        h = og * jnp.tanh(c)
    hr = h.reshape(_NB, 2 * HID)
    hm = (hr[:, :HID] + hr[:, HID:]) * 0.5
    av = a[0, 0]
    hp = jnp.where(hm > 0, hm, av * hm)
    xw = jnp.dot(hp, cw[:], preferred_element_type=F32)
    dinv = lax.rsqrt(c0[:, :1] + c1[:, :1] + 1.0)
    out[:] = xw * dinv


_lstm_call = pl.pallas_call(
    _lstm_body,
    grid=(_GRID,),
    in_specs=[pl.BlockSpec((_PB, NFEAT), functools.partial(lambda i, t: (t * _GRID + i, 0), t=t))
              for t in range(T)]
    + [
        pl.BlockSpec((NFEAT, 4 * HID), lambda i: (0, 0)),
        pl.BlockSpec((HID, 4 * HID), lambda i: (0, 0)),
        pl.BlockSpec((1, 4 * HID), lambda i: (0, 0)),
        pl.BlockSpec((HID, CONV), lambda i: (0, 0)),
        pl.BlockSpec((_NB, 16), lambda i: (i, 0)),
        pl.BlockSpec((_NB, 16), lambda i: (i, 0)),
        pl.BlockSpec(memory_space=pltpu.SMEM),
    ],
    out_specs=pl.BlockSpec((_NB, CONV), lambda i: (i, 0)),
    out_shape=jax.ShapeDtypeStruct((NP, CONV), F32),
)


def _fin_body(s0, s1, xw, c0, c1, b, out):
    acc = s0[:] + s1[:] + xw[:]
    dinv = lax.rsqrt(c0[:, :1] + c1[:, :1] + 1.0)
    v = acc * dinv + b[:]
    n = jnp.sqrt(jnp.sum(v * v, axis=1, keepdims=True))
    out[:] = v / jnp.maximum(n, 1e-12)


_fin_call = pl.pallas_call(
    _fin_body,
    grid=(4,),
    in_specs=[
        pl.BlockSpec((1024, CONV), lambda i: (i, 0)),
        pl.BlockSpec((1024, CONV), lambda i: (i, 0)),
        pl.BlockSpec((1024, CONV), lambda i: (i, 0)),
        pl.BlockSpec((1024, 16), lambda i: (i, 0)),
        pl.BlockSpec((1024, 16), lambda i: (i, 0)),
        pl.BlockSpec((1, CONV), lambda i: (0, 0)),
    ],
    out_specs=pl.BlockSpec((1024, CONV), lambda i: (i, 0)),
    out_shape=jax.ShapeDtypeStruct((NP, CONV), F32),
)


_LB = 512


def _score_body(ga, gb, w1a, w1b, b1, w2, b2, a, out):
    h = (jnp.dot(ga[:], w1a[:], preferred_element_type=F32)
         + jnp.dot(gb[:], w1b[:], preferred_element_type=F32) + b1[:])
    av = a[0, 0]
    h = jnp.where(h > 0, h, av * h)
    out[:] = jnp.dot(h, w2[:], preferred_element_type=F32) + b2[0, 0]


_score_call = pl.pallas_call(
    _score_body,
    grid=(LP // _LB,),
    in_specs=[
        pl.BlockSpec((_LB, 256), lambda i: (i, 0)),
        pl.BlockSpec((_LB, 256), lambda i: (i + LP // _LB, 0)),
        pl.BlockSpec((256, 256), lambda i: (0, 0)),
        pl.BlockSpec((256, 256), lambda i: (0, 0)),
        pl.BlockSpec((1, 256), lambda i: (0, 0)),
        pl.BlockSpec((256, 1), lambda i: (0, 0)),
        pl.BlockSpec(memory_space=pltpu.SMEM),
        pl.BlockSpec(memory_space=pltpu.SMEM),
    ],
    out_specs=pl.BlockSpec((_LB, 1), lambda i: (i, 0)),
    out_shape=jax.ShapeDtypeStruct((LP, 1), F32),
)


def kernel(x, edge_index, edge_attr, all_node_features, rel_features, paths, links,
           x_lin2_W, x_lin2_b, nn_cd_W, nn_cd_b, lstm_Wih, lstm_Whh, lstm_b,
           conv_W, conv_b, lin1_W, lin1_b, lin2_W, lin2_b, prelu_a):
    x_all = jnp.concatenate([all_node_features, rel_features], axis=0)
    src = edge_index[0]
    dst = edge_index[1]
    srcp = jnp.concatenate([src, jnp.zeros((EP - E,), I32)]).reshape(NW, -1, 128)
    dstp = jnp.concatenate([dst, jnp.full((EP - E,), DUMMY, I32)]).reshape(NW, -1, 128)
    pp = jnp.concatenate([paths, jnp.zeros((PP - P, T), I32)], axis=0)
    pidx = pp.T.reshape(NW, -1, 128)
    zeros64 = jnp.zeros((SH, CONV), F32)
    zeros16 = jnp.zeros((SH, 16), F32)
    ones16 = jnp.zeros((128, 16), F32).at[:, 0].set(1.0)
    a2 = prelu_a.reshape(1, 1)

    cnt = _deg_hist(dstp, zeros16, ones16)
    c0 = cnt[0, :NP]
    c1 = cnt[1, :NP]

    states = []
    for i in range(4):
        feats = _gather_paths(x_all, pidx)
        xwp = _lstm_call(*[feats] * T, lstm_Wih[i], lstm_Whh[i],
                         lstm_b[i].reshape(1, -1), conv_W[i], c0, c1, a2)
        S = _edge_scatter(xwp, srcp, dstp, zeros64)
        xc4 = _fin_call(S[0, :NP], S[1, :NP], xwp, c0, c1, conv_b[i].reshape(1, -1))
        states.append(xc4)
        x_pad = jnp.pad(xc4[:N_NODES], ((0, 0), (0, NFEAT - CONV)))
        x_all = x_all.at[2 * N_NODES:3 * N_NODES].set(x_pad)

    cs4 = jnp.concatenate(states, axis=1)
    cs = cs4[:N_NODES]

    l0 = jnp.pad(links[0], (0, LP - L))
    l1 = jnp.pad(links[1], (0, LP - L)) + 2000
    lidx = jnp.concatenate([l0, l1]).reshape(NW, -1, 128)
    g = _gather_links(cs4, lidx)
    outp = _score_call(g, g, lin1_W[:256], lin1_W[256:], lin1_b.reshape(1, -1),
                       lin2_W, lin2_b.reshape(1, 1), a2)
    out = outp[:L, 0]
    return (out, cs, x_all)
